# baseline (speedup 1.0000x reference)
import functools
import math

import jax
import jax.numpy as jnp
from jax import lax
from jax.experimental import pallas as pl
from jax.experimental.pallas import tpu as pltpu

_EPS = 1e-6
_LOG_EPS = math.log(_EPS)
_LOG_1M_EPS = math.log(1.0 - _EPS)


def _fused_kernel(x_ref, hp_ref, w_ref, b_ref, prop_ref, ent_ref,
                  wtop_ref, wbd_ref, acc_ref, hi_ref, *, n_pairs, c, e_dim):
    bb = x_ref.shape[0]
    items = 2 * n_pairs
    lane = lax.broadcasted_iota(jnp.int32, (bb, 2 * c), 1)
    mask = lane < c
    lanelocf = (lane & (c - 1)).astype(jnp.float32)
    ninf = jnp.float32(-jnp.inf)
    dn = (((1,), (0,)), ((), ()))

    def mm(a, b):
        return lax.dot_general(a, b, dn, precision=lax.Precision.DEFAULT,
                               preferred_element_type=jnp.float32)

    @pl.when(pl.program_id(0) == 0)
    def _prep():
        for k in range(n_pairs):
            i0, i1 = 2 * k, 2 * k + 1
            wtop_ref[:, c * i0:c * (i0 + 1)] = w_ref[i0, :e_dim, :]
            wtop_ref[:, c * i1:c * (i1 + 1)] = w_ref[i1, :e_dim, :]
            z = jnp.zeros((c, c), jnp.bfloat16)
            q0 = (w_ref[i0, e_dim:, :]
                  + b_ref[k, 0:1, :].astype(jnp.bfloat16))
            q1 = (w_ref[i1, e_dim:, :]
                  + b_ref[k, 1:2, :].astype(jnp.bfloat16))
            wbd_ref[k, :c, :c] = q0
            wbd_ref[k, :c, c:] = z
            wbd_ref[k, c:, :c] = z
            wbd_ref[k, c:, c:] = q1

    acc_ref[...] = mm(x_ref[...], wtop_ref[...])
    for k in range(n_pairs):
        sl = pl.ds(2 * c * k, 2 * c)
        h0 = hp_ref[:, 2 * k:2 * k + 1]
        h1 = hp_ref[:, 2 * k + 1:2 * k + 2]
        oh = (lane == jnp.where(mask, h0, h1 + c)).astype(jnp.bfloat16)
        acc = acc_ref[:, sl] + mm(oh, wbd_ref[k])
        ma = jnp.max(jnp.where(mask, acc, ninf), axis=1, keepdims=True)
        mb = jnp.max(jnp.where(mask, ninf, acc), axis=1, keepdims=True)
        hit = (acc == jnp.where(mask, ma, mb)).astype(jnp.float32)
        acc_ref[:, sl] = acc
        hi_ref[:, sl] = hit * lanelocf

    g = (lax.broadcasted_iota(jnp.int32, (items * c, items), 0) // c
         == lax.broadcasted_iota(jnp.int32, (items * c, items), 1)
         ).astype(jnp.float32)
    gt = (lax.broadcasted_iota(jnp.int32, (items, items * c), 1) // c
          == lax.broadcasted_iota(jnp.int32, (items, items * c), 0)
          ).astype(jnp.float32)
    acc_all = acc_ref[...]
    e_all = jnp.exp(acc_all)
    s26 = mm(e_all, g)
    i26 = mm(hi_ref[...], g)
    sinv = mm(1.0 / s26, gt)
    lsum = mm(jnp.log(s26), gt)
    p = jnp.clip(e_all * sinv, _EPS, 1.0 - _EPS)
    lp = jnp.clip(acc_all - lsum, _LOG_EPS, _LOG_1M_EPS)
    ent26 = mm(p * lp, g)
    prop_ref[...] = i26.astype(jnp.int32)

    @pl.when(pl.program_id(0) == 0)
    def _init():
        ent_ref[...] = jnp.zeros((1, 1, 1), jnp.float32)

    ent_ref[...] += jnp.reshape(-jnp.sum(ent26), (1, 1, 1))


def kernel(x, hidden_proposal, W, b, testing):
    batch, e_dim = x.shape
    items, ec, c = W.shape
    n_pairs = items // 2
    blk_b = 1024
    hp = hidden_proposal.astype(jnp.int32)

    prop, ent = pl.pallas_call(
        functools.partial(_fused_kernel, n_pairs=n_pairs, c=c, e_dim=e_dim),
        grid=(batch // blk_b,),
        in_specs=[
            pl.BlockSpec((blk_b, e_dim), lambda i: (i, 0)),
            pl.BlockSpec((blk_b, items), lambda i: (i, 0)),
            pl.BlockSpec((items, ec, c), lambda i: (0, 0, 0)),
            pl.BlockSpec((n_pairs, 2, c), lambda i: (0, 0, 0)),
        ],
        out_specs=[
            pl.BlockSpec((blk_b, items), lambda i: (i, 0)),
            pl.BlockSpec((1, 1, 1), lambda i: (0, 0, 0)),
        ],
        out_shape=[
            jax.ShapeDtypeStruct((batch, items), jnp.int32),
            jax.ShapeDtypeStruct((1, 1, 1), jnp.float32),
        ],
        scratch_shapes=[
            pltpu.VMEM((e_dim, items * c), jnp.bfloat16),
            pltpu.VMEM((n_pairs, 2 * c, 2 * c), jnp.bfloat16),
            pltpu.VMEM((blk_b, items * c), jnp.float32),
            pltpu.VMEM((blk_b, items * c), jnp.float32),
        ],
    )(x.astype(jnp.bfloat16), hp, W.astype(jnp.bfloat16),
      b.reshape(n_pairs, 2, c))

    proposal = prop.astype(jnp.int64)
    entropy = ent[0, 0, 0]
    matches = jnp.int32(batch * items)
    draws = jnp.int32(batch * items)
    return (proposal, entropy, matches, draws)


# speedup vs baseline: 1.1161x; 1.1161x over previous
import functools
import math

import jax
import jax.numpy as jnp
from jax import lax
from jax.experimental import pallas as pl
from jax.experimental.pallas import tpu as pltpu

_EPS = 1e-6
_LOG_EPS = math.log(_EPS)
_LOG_1M_EPS = math.log(1.0 - _EPS)


def _fused_kernel(x_ref, hp_ref, w_ref, b_ref, prop_ref, ent_ref,
                  wtop_ref, wbd_ref, acc_ref, hi_ref, *, n_pairs, c, e_dim):
    bb = x_ref.shape[0]
    items = 2 * n_pairs
    lane = lax.broadcasted_iota(jnp.int32, (bb, 2 * c), 1)
    mask = lane < c
    lanelocf = (lane & (c - 1)).astype(jnp.float32)
    ninf = jnp.float32(-jnp.inf)
    dn = (((1,), (0,)), ((), ()))

    def mm(a, b):
        return lax.dot_general(a, b, dn, precision=lax.Precision.DEFAULT,
                               preferred_element_type=jnp.float32)

    @pl.when(pl.program_id(0) == 0)
    def _prep():
        for k in range(n_pairs):
            i0, i1 = 2 * k, 2 * k + 1
            wtop_ref[:, c * i0:c * (i0 + 1)] = w_ref[i0, :e_dim, :]
            wtop_ref[:, c * i1:c * (i1 + 1)] = w_ref[i1, :e_dim, :]
            z = jnp.zeros((c, c), jnp.bfloat16)
            q0 = (w_ref[i0, e_dim:, :]
                  + b_ref[k, 0:1, :].astype(jnp.bfloat16))
            q1 = (w_ref[i1, e_dim:, :]
                  + b_ref[k, 1:2, :].astype(jnp.bfloat16))
            wbd_ref[k, :c, :c] = q0
            wbd_ref[k, :c, c:] = z
            wbd_ref[k, c:, :c] = z
            wbd_ref[k, c:, c:] = q1

    x_blk = x_ref[...].astype(jnp.bfloat16)
    acc_ref[...] = mm(x_blk, wtop_ref[...])
    for k in range(n_pairs):
        sl = pl.ds(2 * c * k, 2 * c)
        h0 = hp_ref[:, 2 * k:2 * k + 1]
        h1 = hp_ref[:, 2 * k + 1:2 * k + 2]
        oh = (lane == jnp.where(mask, h0, h1 + c)).astype(jnp.bfloat16)
        acc = acc_ref[:, sl] + mm(oh, wbd_ref[k])
        ma = jnp.max(jnp.where(mask, acc, ninf), axis=1, keepdims=True)
        mb = jnp.max(jnp.where(mask, ninf, acc), axis=1, keepdims=True)
        hit = (acc == jnp.where(mask, ma, mb)).astype(jnp.float32)
        acc_ref[:, sl] = acc
        hi_ref[:, sl] = hit * lanelocf

    g = (lax.broadcasted_iota(jnp.int32, (items * c, items), 0) // c
         == lax.broadcasted_iota(jnp.int32, (items * c, items), 1)
         ).astype(jnp.float32)
    gt = (lax.broadcasted_iota(jnp.int32, (items, items * c), 1) // c
          == lax.broadcasted_iota(jnp.int32, (items, items * c), 0)
          ).astype(jnp.float32)
    acc_all = acc_ref[...]
    e_all = jnp.exp(acc_all)
    s26 = mm(e_all, g)
    i26 = mm(hi_ref[...], g)
    sinv = mm(1.0 / s26, gt)
    lsum = mm(jnp.log(s26), gt)
    p = jnp.clip(e_all * sinv, _EPS, 1.0 - _EPS)
    lp = jnp.clip(acc_all - lsum, _LOG_EPS, _LOG_1M_EPS)
    ent26 = mm(p * lp, g)
    prop_ref[...] = i26.astype(jnp.int32)

    @pl.when(pl.program_id(0) == 0)
    def _init():
        ent_ref[...] = jnp.zeros((1, 1, 1), jnp.float32)

    ent_ref[...] += jnp.reshape(-jnp.sum(ent26), (1, 1, 1))


def kernel(x, hidden_proposal, W, b, testing):
    batch, e_dim = x.shape
    items, ec, c = W.shape
    n_pairs = items // 2
    blk_b = 1024
    hp = hidden_proposal.astype(jnp.int32)

    prop, ent = pl.pallas_call(
        functools.partial(_fused_kernel, n_pairs=n_pairs, c=c, e_dim=e_dim),
        grid=(batch // blk_b,),
        in_specs=[
            pl.BlockSpec((blk_b, e_dim), lambda i: (i, 0)),
            pl.BlockSpec((blk_b, items), lambda i: (i, 0)),
            pl.BlockSpec((items, ec, c), lambda i: (0, 0, 0)),
            pl.BlockSpec((n_pairs, 2, c), lambda i: (0, 0, 0)),
        ],
        out_specs=[
            pl.BlockSpec((blk_b, items), lambda i: (i, 0)),
            pl.BlockSpec((1, 1, 1), lambda i: (0, 0, 0)),
        ],
        out_shape=[
            jax.ShapeDtypeStruct((batch, items), jnp.int32),
            jax.ShapeDtypeStruct((1, 1, 1), jnp.float32),
        ],
        scratch_shapes=[
            pltpu.VMEM((e_dim, items * c), jnp.bfloat16),
            pltpu.VMEM((n_pairs, 2 * c, 2 * c), jnp.bfloat16),
            pltpu.VMEM((blk_b, items * c), jnp.float32),
            pltpu.VMEM((blk_b, items * c), jnp.float32),
        ],
    )(x, hp, W.astype(jnp.bfloat16), b.reshape(n_pairs, 2, c))

    proposal = prop.astype(jnp.int64)
    entropy = ent[0, 0, 0]
    matches = jnp.int32(batch * items)
    draws = jnp.int32(batch * items)
    return (proposal, entropy, matches, draws)
